# Initial kernel scaffold; baseline (speedup 1.0000x reference)
import jax
import jax.numpy as jnp
from jax.experimental import pallas as pl


def kernel(x, edge_index, edge_attr, params):
    raise NotImplementedError("write your pallas kernel here")


# speedup vs baseline: 2.3009x; 2.3009x over previous
import functools

import jax
import jax.numpy as jnp
from jax import lax
from jax.experimental import pallas as pl
from jax.experimental.pallas import tpu as pltpu
from jax.experimental.pallas import tpu_sc as plsc

N = 10000
E = 320000
D = 128
DE = 16
NB = 3

NC = 2
NS = 16
L = 16
NW = NC * NS

CH = 128
E_PAD = 327680
EPT = E_PAD // NW
NBLK = EPT // CH
EPS2 = E_PAD // NS
NBLK2 = EPS2 // CH
N_DEN = 10240
NPC = N // NC
SPR = 5128
DUMP = 5120
RPT = 320
_NEG = -1e30

_sc_mesh = plsc.VectorSubcoreMesh(
    core_axis_name="c", subcore_axis_name="s", num_cores=NC, num_subcores=NS
)
_sc_params = pltpu.CompilerParams(needs_layout_passes=False)


def _ln(y, g, b):
  mu = jnp.mean(y, axis=-1, keepdims=True)
  var = jnp.mean((y - mu) ** 2, axis=-1, keepdims=True)
  return (y - mu) / jnp.sqrt(var + 1e-5) * g + b


def _enc_body(x, w1, b1, w2, b2, g, b, wl, bl, wr, br, y_o, xl_o, xr_o):
  h = jnp.maximum(x[...] @ w1[...] + b1[...], 0.0)
  y = h @ w2[...] + b2[...]
  y = _ln(y, g[...], b[...])
  y_o[...] = y
  xl_o[...] = y @ wl[...] + bl[...]
  xr_o[...] = y @ wr[...] + br[...]


def _eproj_body(ea, w0, w1, w2, o0, o1, o2):
  a = ea[...]
  o0[...] = a @ w0[...]
  o1[...] = a @ w1[...]
  o2[...] = a @ w2[...]


def _comb_body(p, y, bias, g, b, wl, bl, wr, br, y_o, xl_o, xr_o):
  o = p[0] + bias[...]
  y2 = y[...] + _ln(o, g[...], b[...])
  y_o[...] = y2
  xl_o[...] = y2 @ wl[...] + bl[...]
  xr_o[...] = y2 @ wr[...] + br[...]


def _final_body(p, y, bias, g, b, w1, b1, w2, b2, out_o):
  o = p[0] + bias[...]
  y2 = y[...] + _ln(o, g[...], b[...])
  h = jnp.maximum(y2 @ w1[...] + b1[...], 0.0)
  out_o[...] = h @ w2[...] + b2[...]


_ROWB = 1000


def _row_spec():
  return pl.BlockSpec((_ROWB, D), lambda i: (i, 0))


def _full_spec(shape):
  n = len(shape)
  return pl.BlockSpec(shape, lambda i: (0,) * n)


def _part_spec():
  return pl.BlockSpec((1, _ROWB, D), lambda i: (i // 5, i % 5, 0))


def _tc_enc(x, enc, gnn0):
  return pl.pallas_call(
      _enc_body,
      grid=(N // _ROWB,),
      in_specs=[
          _row_spec(),
          _full_spec((D, D)), _full_spec((1, D)),
          _full_spec((D, D)), _full_spec((1, D)),
          _full_spec((1, D)), _full_spec((1, D)),
          _full_spec((D, D)), _full_spec((1, D)),
          _full_spec((D, D)), _full_spec((1, D)),
      ],
      out_specs=[_row_spec(), _row_spec(), _row_spec()],
      out_shape=[jax.ShapeDtypeStruct((N, D), jnp.float32)] * 3,
  )(x, enc['W1'], enc['b1'].reshape(1, D), enc['W2'], enc['b2'].reshape(1, D),
    enc['ln_g'].reshape(1, D), enc['ln_b'].reshape(1, D),
    gnn0['Wl'], gnn0['bl'].reshape(1, D), gnn0['Wr'], gnn0['br'].reshape(1, D))


_EB = 4096


def _tc_eproj(ea_pad, we):
  espec = pl.BlockSpec((_EB, DE), lambda i: (i, 0))
  ospec = pl.BlockSpec((_EB, D), lambda i: (i, 0))
  return pl.pallas_call(
      _eproj_body,
      grid=(E_PAD // _EB,),
      in_specs=[espec] + [_full_spec((DE, D))] * 3,
      out_specs=[ospec] * 3,
      out_shape=[jax.ShapeDtypeStruct((E_PAD, D), jnp.float32)] * 3,
  )(ea_pad, we[0], we[1], we[2])


def _tc_comb(parts, y, p_cur, p_nxt):
  return pl.pallas_call(
      _comb_body,
      grid=(N // _ROWB,),
      in_specs=[
          _part_spec(), _row_spec(),
          _full_spec((1, D)), _full_spec((1, D)), _full_spec((1, D)),
          _full_spec((D, D)), _full_spec((1, D)),
          _full_spec((D, D)), _full_spec((1, D)),
      ],
      out_specs=[_row_spec(), _row_spec(), _row_spec()],
      out_shape=[jax.ShapeDtypeStruct((N, D), jnp.float32)] * 3,
  )(parts, y, p_cur['bias'].reshape(1, D), p_cur['ln_g'].reshape(1, D),
    p_cur['ln_b'].reshape(1, D),
    p_nxt['Wl'], p_nxt['bl'].reshape(1, D), p_nxt['Wr'],
    p_nxt['br'].reshape(1, D))


def _tc_final(parts, y, p_cur, dec):
  return pl.pallas_call(
      _final_body,
      grid=(N // _ROWB,),
      in_specs=[
          _part_spec(), _row_spec(),
          _full_spec((1, D)), _full_spec((1, D)), _full_spec((1, D)),
          _full_spec((D, D)), _full_spec((1, D)),
          _full_spec((D, D)), _full_spec((1, D)),
      ],
      out_specs=_row_spec(),
      out_shape=jax.ShapeDtypeStruct((N, D), jnp.float32),
  )(parts, y, p_cur['bias'].reshape(1, D), p_cur['ln_g'].reshape(1, D),
    p_cur['ln_b'].reshape(1, D),
    dec['W1'], dec['b1'].reshape(1, D), dec['W2'], dec['b2'].reshape(1, D))


def _lane_perm(v, idx):
  return jnp.take_along_axis(v, idx, axis=0)


def _bfly_sum(v):
  iota = lax.iota(jnp.int32, L)
  for sh in (8, 4, 2, 1):
    v = v + _lane_perm(v, jnp.bitwise_xor(iota, sh))
  return v


def _bfly_max(v):
  iota = lax.iota(jnp.int32, L)
  for sh in (8, 4, 2, 1):
    v = jnp.maximum(v, _lane_perm(v, jnp.bitwise_xor(iota, sh)))
  return v


def _sc1_body(xl_hbm, xr_hbm, ep_hbm, src_hbm, dst_hbm, att_hbm, logits_hbm,
              sidx, didx, att_v, xlb, xrb, epb, lb, sem):
  cid = lax.axis_index("c")
  sid = lax.axis_index("s")
  tid = sid * NC + cid
  base = tid * EPT
  pltpu.sync_copy(att_hbm, att_v)
  att_c = [att_v[pl.ds(c * L, L)] for c in range(D // L)]
  iota = lax.iota(jnp.int32, L)

  def blk(b, _):
    off = b * CH
    pltpu.sync_copy(src_hbm.at[pl.ds(base + off, CH)], sidx)
    pltpu.sync_copy(dst_hbm.at[pl.ds(base + off, CH)], didx)
    cp1 = pltpu.async_copy(xl_hbm.at[sidx], xlb, sem)
    cp2 = pltpu.async_copy(xr_hbm.at[didx], xrb, sem)
    cp3 = pltpu.async_copy(ep_hbm.at[pl.ds(base + off, CH)], epb, sem)
    cp1.wait()
    cp2.wait()
    cp3.wait()

    def grp(g, _):
      acc = jnp.zeros((L,), jnp.float32)
      for j in range(L):
        e = g * L + j
        dot = jnp.zeros((L,), jnp.float32)
        for c in range(D // L):
          z = (xlb[e, pl.ds(c * L, L)] + xrb[e, pl.ds(c * L, L)]
               + epb[e, pl.ds(c * L, L)])
          z = jnp.maximum(z, 0.2 * z)
          dot = dot + z * att_c[c]
        svec = _bfly_sum(dot)
        ge = base + off + e
        svec = jnp.where(ge < E, svec, jnp.full((L,), _NEG, jnp.float32))
        acc = jnp.where(iota == j, svec, acc)
      lb[pl.ds(g * L, L)] = acc
      return 0

    lax.fori_loop(0, CH // L, grp, 0)
    pltpu.sync_copy(lb, logits_hbm.at[pl.ds(base + off, CH)])
    return 0

  lax.fori_loop(0, NBLK, blk, 0)


def _sc1(xl, xr, ep, src_pad, dst_pad, att):
  f = pl.kernel(
      _sc1_body,
      out_type=jax.ShapeDtypeStruct((E_PAD,), jnp.float32),
      mesh=_sc_mesh,
      compiler_params=_sc_params,
      scratch_types=[
          pltpu.VMEM((CH,), jnp.int32),
          pltpu.VMEM((CH,), jnp.int32),
          pltpu.VMEM((D,), jnp.float32),
          pltpu.VMEM((CH, D), jnp.float32),
          pltpu.VMEM((CH, D), jnp.float32),
          pltpu.VMEM((CH, D), jnp.float32),
          pltpu.VMEM((CH,), jnp.float32),
          pltpu.SemaphoreType.DMA,
      ],
  )
  return f(xl, xr, ep, src_pad, dst_pad, att)


def _sc2_body(logits_hbm, src_hbm, dst_hbm, xl_hbm, part_hbm,
              den_v, lrange, drange, mxrow, mxbuf, exb, dch,
              sch, dch2, wb, xlb, wrow,
              spmem_den, spmem_out, spmem_mx):
  cid = lax.axis_index("c")
  sid = lax.axis_index("s")

  def zrow(i, _):
    for c in range(D // L):
      wrow[i, pl.ds(c * L, L)] = jnp.zeros((L,), jnp.float32)
    return 0
  lax.fori_loop(0, CH, zrow, 0)
  for k in range(RPT // 64):
    pltpu.sync_copy(wrow.at[pl.ds(0, 64)],
                    spmem_out.at[pl.ds(sid * RPT + k * 64, 64)])

  @pl.when(sid == NS - 1)
  def _():
    pltpu.sync_copy(wrow.at[pl.ds(0, SPR - NS * RPT)],
                    spmem_out.at[pl.ds(NS * RPT, SPR - NS * RPT)])

  @pl.when(sid == 0)
  def _():
    def zden(i, _):
      den_v[pl.ds(i * L, L)] = jnp.zeros((L,), jnp.float32)
      return 0
    lax.fori_loop(0, N_DEN // L, zden, 0)
    pltpu.sync_copy(den_v, spmem_den)

  plsc.subcore_barrier()

  base2 = sid * EPS2
  pltpu.sync_copy(logits_hbm.at[pl.ds(base2, EPS2)], lrange)
  pltpu.sync_copy(dst_hbm.at[pl.ds(base2, EPS2)], drange)

  def mx_step(g, m):
    return jnp.maximum(m, lrange[pl.ds(g * L, L)])
  m = lax.fori_loop(0, EPS2 // L, mx_step, jnp.full((L,), _NEG, jnp.float32))
  mxrow[pl.ds(0, L)] = m
  pltpu.sync_copy(mxrow, spmem_mx.at[pl.ds(sid * L, L)])
  plsc.subcore_barrier()
  pltpu.sync_copy(spmem_mx, mxbuf)

  def mx_all(i, mm):
    return jnp.maximum(mm, mxbuf[pl.ds(i * L, L)])
  mm = lax.fori_loop(0, NS, mx_all, jnp.full((L,), _NEG, jnp.float32))
  cmax = _bfly_max(mm)

  def den_blk(b, _):
    off = b * CH
    for g in range(CH // L):
      lv = lrange[pl.ds(off + g * L, L)]
      exb[pl.ds(g * L, L)] = jnp.exp(lv - cmax)
      dch[pl.ds(g * L, L)] = drange[pl.ds(off + g * L, L)]
    pltpu.sync_copy(exb, spmem_den.at[dch], add=True)
    return 0
  lax.fori_loop(0, NBLK2, den_blk, 0)
  plsc.subcore_barrier()

  pltpu.sync_copy(spmem_den, den_v)

  row0 = cid * NPC

  def out_blk(b, _):
    off = b * CH
    pltpu.sync_copy(src_hbm.at[pl.ds(base2 + off, CH)], sch)
    pltpu.sync_copy(xl_hbm.at[sch], xlb)
    for g in range(CH // L):
      lv = lrange[pl.ds(off + g * L, L)]
      dv = drange[pl.ds(off + g * L, L)]
      denv = plsc.load_gather(den_v, [dv])
      wb[pl.ds(g * L, L)] = jnp.exp(lv - cmax) / (denv + 1e-16)
      loc = dv - row0
      ok = (loc >= 0) & (loc < NPC)
      dch2[pl.ds(g * L, L)] = jnp.where(ok, loc, DUMP)

    def erow(e, _):
      w16 = plsc.load_gather(wb, [jnp.full((L,), e, jnp.int32)])
      for c in range(D // L):
        wrow[e, pl.ds(c * L, L)] = xlb[e, pl.ds(c * L, L)] * w16
      return 0
    lax.fori_loop(0, CH, erow, 0)
    pltpu.sync_copy(wrow, spmem_out.at[dch2], add=True)
    return 0
  lax.fori_loop(0, NBLK2, out_blk, 0)
  plsc.subcore_barrier()

  for k in range(RPT // 64):
    r0 = sid * RPT + k * 64
    pltpu.sync_copy(spmem_out.at[pl.ds(r0, 64)],
                    part_hbm.at[pl.ds(cid * NS * RPT + r0, 64)])


def _sc2(logits, src_pad, dst_pad, xl):
  f = pl.kernel(
      _sc2_body,
      out_type=jax.ShapeDtypeStruct((NC * NS * RPT, D), jnp.float32),
      mesh=_sc_mesh,
      compiler_params=_sc_params,
      scratch_types=[
          pltpu.VMEM((N_DEN,), jnp.float32),
          pltpu.VMEM((EPS2,), jnp.float32),
          pltpu.VMEM((EPS2,), jnp.int32),
          pltpu.VMEM((L,), jnp.float32),
          pltpu.VMEM((NS * L,), jnp.float32),
          pltpu.VMEM((CH,), jnp.float32),
          pltpu.VMEM((CH,), jnp.int32),
          pltpu.VMEM((CH,), jnp.int32),
          pltpu.VMEM((CH,), jnp.int32),
          pltpu.VMEM((CH,), jnp.float32),
          pltpu.VMEM((CH, D), jnp.float32),
          pltpu.VMEM((CH, D), jnp.float32),
          pltpu.VMEM_SHARED((N_DEN,), jnp.float32),
          pltpu.VMEM_SHARED((SPR, D), jnp.float32),
          pltpu.VMEM_SHARED((NS * L,), jnp.float32),
      ],
  )
  return f(logits, src_pad, dst_pad, xl).reshape(NC, NS * RPT, D)


def kernel(x, edge_index, edge_attr, params):
  src = edge_index[0].astype(jnp.int32)
  dst = edge_index[1].astype(jnp.int32)
  pad = E_PAD - E
  src_pad = jnp.concatenate([src, jnp.zeros((pad,), jnp.int32)])
  dst_pad = jnp.concatenate([dst, jnp.zeros((pad,), jnp.int32)])
  ea_pad = jnp.concatenate(
      [edge_attr, jnp.zeros((pad, DE), jnp.float32)], axis=0)

  enc = params['enc']
  gnn = params['gnn']
  dec = params['dec']

  y, xl, xr = _tc_enc(x, enc, gnn[0])
  eps = _tc_eproj(ea_pad, [gnn[0]['We'], gnn[1]['We'], gnn[2]['We']])

  for i in range(NB):
    p = gnn[i]
    logits = _sc1(xl, xr, eps[i], src_pad, dst_pad, p['att'])
    parts = _sc2(logits, src_pad, dst_pad, xl)
    if i + 1 < NB:
      y, xl, xr = _tc_comb(parts, y, p, gnn[i + 1])
    else:
      out = _tc_final(parts, y, p, dec)
  return out


# speedup vs baseline: 3.3676x; 1.4636x over previous
import functools

import jax
import jax.numpy as jnp
from jax import lax
from jax.experimental import pallas as pl
from jax.experimental.pallas import tpu as pltpu
from jax.experimental.pallas import tpu_sc as plsc

N = 10000
E = 320000
D = 128
DE = 16
NB = 3

NC = 2
NS = 16
L = 16
NW = NC * NS

CH = 128
E_PAD = 327680
EPT = E_PAD // NW
NBLK = EPT // CH
EPS2 = E_PAD // NS
NBLK2 = EPS2 // CH
N_DEN = 10240
NPC = N // NC
SPR = 5128
DUMP = 5120
RPT = 320
_NEG = -1e30

_sc_mesh = plsc.VectorSubcoreMesh(
    core_axis_name="c", subcore_axis_name="s", num_cores=NC, num_subcores=NS
)
_sc_params = pltpu.CompilerParams(needs_layout_passes=False)


def _ln(y, g, b):
  mu = jnp.mean(y, axis=-1, keepdims=True)
  var = jnp.mean((y - mu) ** 2, axis=-1, keepdims=True)
  return (y - mu) / jnp.sqrt(var + 1e-5) * g + b


def _enc_body(x, w1, b1, w2, b2, g, b, wl, bl, wr, br, y_o, xl_o, xr_o):
  h = jnp.maximum(x[...] @ w1[...] + b1[...], 0.0)
  y = h @ w2[...] + b2[...]
  y = _ln(y, g[...], b[...])
  y_o[...] = y
  xl_o[...] = y @ wl[...] + bl[...]
  xr_o[...] = y @ wr[...] + br[...]


def _eproj_body(ea, w0, w1, w2, o0, o1, o2):
  a = ea[...]
  o0[...] = a @ w0[...]
  o1[...] = a @ w1[...]
  o2[...] = a @ w2[...]


def _comb_body(p, y, bias, g, b, wl, bl, wr, br, y_o, xl_o, xr_o):
  o = p[0] + bias[...]
  y2 = y[...] + _ln(o, g[...], b[...])
  y_o[...] = y2
  xl_o[...] = y2 @ wl[...] + bl[...]
  xr_o[...] = y2 @ wr[...] + br[...]


def _final_body(p, y, bias, g, b, w1, b1, w2, b2, out_o):
  o = p[0] + bias[...]
  y2 = y[...] + _ln(o, g[...], b[...])
  h = jnp.maximum(y2 @ w1[...] + b1[...], 0.0)
  out_o[...] = h @ w2[...] + b2[...]


_ROWB = 1000


def _row_spec():
  return pl.BlockSpec((_ROWB, D), lambda i: (i, 0))


def _full_spec(shape):
  n = len(shape)
  return pl.BlockSpec(shape, lambda i: (0,) * n)


def _part_spec():
  return pl.BlockSpec((1, _ROWB, D), lambda i: (i // 5, i % 5, 0))


def _tc_enc(x, enc, gnn0):
  return pl.pallas_call(
      _enc_body,
      grid=(N // _ROWB,),
      in_specs=[
          _row_spec(),
          _full_spec((D, D)), _full_spec((1, D)),
          _full_spec((D, D)), _full_spec((1, D)),
          _full_spec((1, D)), _full_spec((1, D)),
          _full_spec((D, D)), _full_spec((1, D)),
          _full_spec((D, D)), _full_spec((1, D)),
      ],
      out_specs=[_row_spec(), _row_spec(), _row_spec()],
      out_shape=[jax.ShapeDtypeStruct((N, D), jnp.float32)] * 3,
  )(x, enc['W1'], enc['b1'].reshape(1, D), enc['W2'], enc['b2'].reshape(1, D),
    enc['ln_g'].reshape(1, D), enc['ln_b'].reshape(1, D),
    gnn0['Wl'], gnn0['bl'].reshape(1, D), gnn0['Wr'], gnn0['br'].reshape(1, D))


_EB = 4096


def _tc_eproj(ea_pad, we):
  espec = pl.BlockSpec((_EB, DE), lambda i: (i, 0))
  ospec = pl.BlockSpec((_EB, D), lambda i: (i, 0))
  return pl.pallas_call(
      _eproj_body,
      grid=(E_PAD // _EB,),
      in_specs=[espec] + [_full_spec((DE, D))] * 3,
      out_specs=[ospec] * 3,
      out_shape=[jax.ShapeDtypeStruct((E_PAD, D), jnp.float32)] * 3,
  )(ea_pad, we[0], we[1], we[2])


def _tc_comb(parts, y, p_cur, p_nxt):
  return pl.pallas_call(
      _comb_body,
      grid=(N // _ROWB,),
      in_specs=[
          _part_spec(), _row_spec(),
          _full_spec((1, D)), _full_spec((1, D)), _full_spec((1, D)),
          _full_spec((D, D)), _full_spec((1, D)),
          _full_spec((D, D)), _full_spec((1, D)),
      ],
      out_specs=[_row_spec(), _row_spec(), _row_spec()],
      out_shape=[jax.ShapeDtypeStruct((N, D), jnp.float32)] * 3,
  )(parts, y, p_cur['bias'].reshape(1, D), p_cur['ln_g'].reshape(1, D),
    p_cur['ln_b'].reshape(1, D),
    p_nxt['Wl'], p_nxt['bl'].reshape(1, D), p_nxt['Wr'],
    p_nxt['br'].reshape(1, D))


def _tc_final(parts, y, p_cur, dec):
  return pl.pallas_call(
      _final_body,
      grid=(N // _ROWB,),
      in_specs=[
          _part_spec(), _row_spec(),
          _full_spec((1, D)), _full_spec((1, D)), _full_spec((1, D)),
          _full_spec((D, D)), _full_spec((1, D)),
          _full_spec((D, D)), _full_spec((1, D)),
      ],
      out_specs=_row_spec(),
      out_shape=jax.ShapeDtypeStruct((N, D), jnp.float32),
  )(parts, y, p_cur['bias'].reshape(1, D), p_cur['ln_g'].reshape(1, D),
    p_cur['ln_b'].reshape(1, D),
    dec['W1'], dec['b1'].reshape(1, D), dec['W2'], dec['b2'].reshape(1, D))


def _lane_perm(v, idx):
  return jnp.take_along_axis(v, idx, axis=0)


def _bfly_sum(v):
  iota = lax.iota(jnp.int32, L)
  for sh in (8, 4, 2, 1):
    v = v + _lane_perm(v, jnp.bitwise_xor(iota, sh))
  return v


def _bfly_max(v):
  iota = lax.iota(jnp.int32, L)
  for sh in (8, 4, 2, 1):
    v = jnp.maximum(v, _lane_perm(v, jnp.bitwise_xor(iota, sh)))
  return v


def _sc1_body(xl_hbm, xr_hbm, ep_hbm, src_hbm, dst_hbm, att_hbm, logits_hbm,
              src_v, dst_v, att_v,
              sidx0, didx0, xlb0, xrb0, epb0, lb0,
              sidx1, didx1, xlb1, xrb1, epb1, lb1,
              sem0, sem1, lsem0, lsem1):
  cid = lax.axis_index("c")
  sid = lax.axis_index("s")
  tid = sid * NC + cid
  base = tid * EPT
  pltpu.sync_copy(src_hbm.at[pl.ds(base, EPT)], src_v)
  pltpu.sync_copy(dst_hbm.at[pl.ds(base, EPT)], dst_v)
  pltpu.sync_copy(att_hbm, att_v)
  att_c = [att_v[pl.ds(c * L, L)] for c in range(D // L)]
  iota = lax.iota(jnp.int32, L)
  BUFS = ((sidx0, didx0, xlb0, xrb0, epb0, lb0, sem0, lsem0),
          (sidx1, didx1, xlb1, xrb1, epb1, lb1, sem1, lsem1))

  def idx_fire(k, b):
    sx, dx, xlb, xrb, epb, _, sem, _ = BUFS[k]
    off = b * CH
    for g in range(CH // L):
      sx[pl.ds(g * L, L)] = src_v[pl.ds(off + g * L, L)]
      dx[pl.ds(g * L, L)] = dst_v[pl.ds(off + g * L, L)]
    pltpu.async_copy(xl_hbm.at[sx], xlb, sem)
    pltpu.async_copy(xr_hbm.at[dx], xrb, sem)
    pltpu.async_copy(ep_hbm.at[pl.ds(base + off, CH)], epb, sem)

  def drain(k, b):
    sx, dx, xlb, xrb, epb, _, sem, _ = BUFS[k]
    pltpu.make_async_copy(xl_hbm.at[sx], xlb, sem).wait()
    pltpu.make_async_copy(xr_hbm.at[dx], xrb, sem).wait()
    pltpu.make_async_copy(ep_hbm.at[pl.ds(base, CH)], epb, sem).wait()

  def compute(k, b):
    _, _, xlb, xrb, epb, lb, _, lsem = BUFS[k]
    off = b * CH

    @pl.when(b >= 2)
    def _():
      pltpu.make_async_copy(lb, logits_hbm.at[pl.ds(base, CH)], lsem).wait()

    def grp(g, _):
      acc = jnp.zeros((L,), jnp.float32)
      for j in range(L):
        e = g * L + j
        dot = jnp.zeros((L,), jnp.float32)
        for c in range(D // L):
          z = (xlb[e, pl.ds(c * L, L)] + xrb[e, pl.ds(c * L, L)]
               + epb[e, pl.ds(c * L, L)])
          z = jnp.maximum(z, 0.2 * z)
          dot = dot + z * att_c[c]
        svec = _bfly_sum(dot)
        ge = base + off + e
        svec = jnp.where(ge < E, svec, jnp.full((L,), _NEG, jnp.float32))
        acc = jnp.where(iota == j, svec, acc)
      lb[pl.ds(g * L, L)] = acc
      return 0

    lax.fori_loop(0, CH // L, grp, 0)
    pltpu.async_copy(lb, logits_hbm.at[pl.ds(base + off, CH)], lsem)

  idx_fire(0, 0)

  def pair(i, _):
    b0 = i * 2
    idx_fire(1, b0 + 1)
    drain(0, b0)
    compute(0, b0)

    @pl.when(b0 + 2 < NBLK)
    def _():
      idx_fire(0, b0 + 2)

    drain(1, b0 + 1)
    compute(1, b0 + 1)
    return 0

  lax.fori_loop(0, NBLK // 2, pair, 0)
  pltpu.make_async_copy(lb0, logits_hbm.at[pl.ds(base, CH)], lsem0).wait()
  pltpu.make_async_copy(lb1, logits_hbm.at[pl.ds(base, CH)], lsem1).wait()


def _sc1(xl, xr, ep, src_pad, dst_pad, att):
  f = pl.kernel(
      _sc1_body,
      out_type=jax.ShapeDtypeStruct((E_PAD,), jnp.float32),
      mesh=_sc_mesh,
      compiler_params=_sc_params,
      scratch_types=[
          pltpu.VMEM((EPT,), jnp.int32),
          pltpu.VMEM((EPT,), jnp.int32),
          pltpu.VMEM((D,), jnp.float32),
          pltpu.VMEM((CH,), jnp.int32),
          pltpu.VMEM((CH,), jnp.int32),
          pltpu.VMEM((CH, D), jnp.float32),
          pltpu.VMEM((CH, D), jnp.float32),
          pltpu.VMEM((CH, D), jnp.float32),
          pltpu.VMEM((CH,), jnp.float32),
          pltpu.VMEM((CH,), jnp.int32),
          pltpu.VMEM((CH,), jnp.int32),
          pltpu.VMEM((CH, D), jnp.float32),
          pltpu.VMEM((CH, D), jnp.float32),
          pltpu.VMEM((CH, D), jnp.float32),
          pltpu.VMEM((CH,), jnp.float32),
          pltpu.SemaphoreType.DMA,
          pltpu.SemaphoreType.DMA,
          pltpu.SemaphoreType.DMA,
          pltpu.SemaphoreType.DMA,
      ],
  )
  return f(xl, xr, ep, src_pad, dst_pad, att)


CH3 = 64
NB3 = EPS2 // CH3


def _sc2_body(logits_hbm, src_hbm, dst_hbm, xl_hbm, part_hbm,
              den_v, lrange, drange, mxrow, mxbuf,
              exb0, dch0, exb1, dch1,
              sch0, dch20, wb0, xlb0, wrow0,
              sch1, dch21, wb1, xlb1, wrow1,
              dsem0, dsem1, gsem0, gsem1, ssem0, ssem1, isem0, isem1,
              spmem_den, spmem_out, spmem_mx):
  cid = lax.axis_index("c")
  sid = lax.axis_index("s")

  def zrow(i, _):
    for c in range(D // L):
      wrow0[i, pl.ds(c * L, L)] = jnp.zeros((L,), jnp.float32)
    return 0
  lax.fori_loop(0, CH3, zrow, 0)
  for k in range(RPT // CH3):
    pltpu.sync_copy(wrow0, spmem_out.at[pl.ds(sid * RPT + k * CH3, CH3)])

  @pl.when(sid == NS - 1)
  def _():
    pltpu.sync_copy(wrow0.at[pl.ds(0, SPR - NS * RPT)],
                    spmem_out.at[pl.ds(NS * RPT, SPR - NS * RPT)])

  @pl.when(sid == 0)
  def _():
    def zden(i, _):
      den_v[pl.ds(i * L, L)] = jnp.zeros((L,), jnp.float32)
      return 0
    lax.fori_loop(0, N_DEN // L, zden, 0)
    pltpu.sync_copy(den_v, spmem_den)

  plsc.subcore_barrier()

  base2 = sid * EPS2
  pltpu.sync_copy(logits_hbm.at[pl.ds(base2, EPS2)], lrange)
  pltpu.sync_copy(dst_hbm.at[pl.ds(base2, EPS2)], drange)

  def mx_step(g, m):
    return jnp.maximum(m, lrange[pl.ds(g * L, L)])
  m = lax.fori_loop(0, EPS2 // L, mx_step, jnp.full((L,), _NEG, jnp.float32))
  mxrow[pl.ds(0, L)] = m
  pltpu.sync_copy(mxrow, spmem_mx.at[pl.ds(sid * L, L)])
  plsc.subcore_barrier()
  pltpu.sync_copy(spmem_mx, mxbuf)

  def mx_all(i, mm):
    return jnp.maximum(mm, mxbuf[pl.ds(i * L, L)])
  mm = lax.fori_loop(0, NS, mx_all, jnp.full((L,), _NEG, jnp.float32))
  cmax = _bfly_max(mm)

  DBUFS = ((exb0, dch0, dsem0), (exb1, dch1, dsem1))

  def den_one(k, b):
    exb, dch, dsem = DBUFS[k]

    @pl.when(b >= 2)
    def _():
      pltpu.make_async_copy(exb, spmem_den.at[dch], dsem).wait()

    off = b * CH
    for g in range(CH // L):
      lv = lrange[pl.ds(off + g * L, L)]
      exb[pl.ds(g * L, L)] = jnp.exp(lv - cmax)
      dch[pl.ds(g * L, L)] = drange[pl.ds(off + g * L, L)]
    pltpu.async_copy(exb, spmem_den.at[dch], dsem, add=True)

  def den_pair(i, _):
    den_one(0, i * 2)
    den_one(1, i * 2 + 1)
    return 0
  lax.fori_loop(0, NBLK2 // 2, den_pair, 0)
  pltpu.make_async_copy(exb0, spmem_den.at[dch0], dsem0).wait()
  pltpu.make_async_copy(exb1, spmem_den.at[dch1], dsem1).wait()
  plsc.subcore_barrier()

  pltpu.sync_copy(spmem_den, den_v)

  row0 = cid * NPC
  OBUFS = ((sch0, dch20, wb0, xlb0, wrow0, gsem0, ssem0, isem0),
           (sch1, dch21, wb1, xlb1, wrow1, gsem1, ssem1, isem1))

  def idx_fire(k, b):
    sch, isem = OBUFS[k][0], OBUFS[k][7]
    pltpu.async_copy(src_hbm.at[pl.ds(base2 + b * CH3, CH3)], sch, isem)

  def out_one(k, b):
    sch, dch2, wb, xlb, wrow, gsem, ssem, isem = OBUFS[k]
    off = b * CH3

    @pl.when(b >= 2)
    def _():
      pltpu.make_async_copy(wrow, spmem_out.at[dch2], ssem).wait()

    @pl.when(b + 1 < NB3)
    def _():
      osch, oxlb = OBUFS[1 - k][0], OBUFS[1 - k][3]
      pltpu.make_async_copy(src_hbm.at[pl.ds(base2, CH3)], osch,
                            OBUFS[1 - k][7]).wait()
      pltpu.async_copy(xl_hbm.at[osch], oxlb, OBUFS[1 - k][5])

    pltpu.make_async_copy(xl_hbm.at[sch], xlb, gsem).wait()

    @pl.when(b + 2 < NB3)
    def _():
      idx_fire(k, b + 2)

    for g in range(CH3 // L):
      lv = lrange[pl.ds(off + g * L, L)]
      dv = drange[pl.ds(off + g * L, L)]
      denv = plsc.load_gather(den_v, [dv])
      wb[pl.ds(g * L, L)] = jnp.exp(lv - cmax) / (denv + 1e-16)
      loc = dv - row0
      ok = (loc >= 0) & (loc < NPC)
      dch2[pl.ds(g * L, L)] = jnp.where(ok, loc, DUMP)

    def erow(e, _):
      w16 = plsc.load_gather(wb, [jnp.full((L,), e, jnp.int32)])
      for c in range(D // L):
        wrow[e, pl.ds(c * L, L)] = xlb[e, pl.ds(c * L, L)] * w16
      return 0
    lax.fori_loop(0, CH3, erow, 0)
    pltpu.async_copy(wrow, spmem_out.at[dch2], ssem, add=True)

  pltpu.sync_copy(src_hbm.at[pl.ds(base2, CH3)], sch0)
  pltpu.async_copy(xl_hbm.at[sch0], xlb0, gsem0)
  idx_fire(1, 1)

  def out_pair(i, _):
    out_one(0, i * 2)
    out_one(1, i * 2 + 1)
    return 0
  lax.fori_loop(0, NB3 // 2, out_pair, 0)
  pltpu.make_async_copy(wrow0, spmem_out.at[dch20], ssem0).wait()
  pltpu.make_async_copy(wrow1, spmem_out.at[dch21], ssem1).wait()
  plsc.subcore_barrier()

  for k in range(RPT // 64):
    r0 = sid * RPT + k * 64
    pltpu.sync_copy(spmem_out.at[pl.ds(r0, 64)],
                    part_hbm.at[pl.ds(cid * NS * RPT + r0, 64)])


def _sc2(logits, src_pad, dst_pad, xl):
  f = pl.kernel(
      _sc2_body,
      out_type=jax.ShapeDtypeStruct((NC * NS * RPT, D), jnp.float32),
      mesh=_sc_mesh,
      compiler_params=_sc_params,
      scratch_types=[
          pltpu.VMEM((N_DEN,), jnp.float32),
          pltpu.VMEM((EPS2,), jnp.float32),
          pltpu.VMEM((EPS2,), jnp.int32),
          pltpu.VMEM((L,), jnp.float32),
          pltpu.VMEM((NS * L,), jnp.float32),
          pltpu.VMEM((CH,), jnp.float32),
          pltpu.VMEM((CH,), jnp.int32),
          pltpu.VMEM((CH,), jnp.float32),
          pltpu.VMEM((CH,), jnp.int32),
          pltpu.VMEM((CH3,), jnp.int32),
          pltpu.VMEM((CH3,), jnp.int32),
          pltpu.VMEM((CH3,), jnp.float32),
          pltpu.VMEM((CH3, D), jnp.float32),
          pltpu.VMEM((CH3, D), jnp.float32),
          pltpu.VMEM((CH3,), jnp.int32),
          pltpu.VMEM((CH3,), jnp.int32),
          pltpu.VMEM((CH3,), jnp.float32),
          pltpu.VMEM((CH3, D), jnp.float32),
          pltpu.VMEM((CH3, D), jnp.float32),
          pltpu.SemaphoreType.DMA,
          pltpu.SemaphoreType.DMA,
          pltpu.SemaphoreType.DMA,
          pltpu.SemaphoreType.DMA,
          pltpu.SemaphoreType.DMA,
          pltpu.SemaphoreType.DMA,
          pltpu.SemaphoreType.DMA,
          pltpu.SemaphoreType.DMA,
          pltpu.VMEM_SHARED((N_DEN,), jnp.float32),
          pltpu.VMEM_SHARED((SPR, D), jnp.float32),
          pltpu.VMEM_SHARED((NS * L,), jnp.float32),
      ],
  )
  return f(logits, src_pad, dst_pad, xl).reshape(NC, NS * RPT, D)


def kernel(x, edge_index, edge_attr, params):
  src = edge_index[0].astype(jnp.int32)
  dst = edge_index[1].astype(jnp.int32)
  pad = E_PAD - E
  src_pad = jnp.concatenate([src, jnp.zeros((pad,), jnp.int32)])
  dst_pad = jnp.concatenate([dst, jnp.zeros((pad,), jnp.int32)])
  ea_pad = jnp.concatenate(
      [edge_attr, jnp.zeros((pad, DE), jnp.float32)], axis=0)

  enc = params['enc']
  gnn = params['gnn']
  dec = params['dec']

  y, xl, xr = _tc_enc(x, enc, gnn[0])
  eps = _tc_eproj(ea_pad, [gnn[0]['We'], gnn[1]['We'], gnn[2]['We']])

  for i in range(NB):
    p = gnn[i]
    logits = _sc1(xl, xr, eps[i], src_pad, dst_pad, p['att'])
    parts = _sc2(logits, src_pad, dst_pad, xl)
    if i + 1 < NB:
      y, xl, xr = _tc_comb(parts, y, p, gnn[i + 1])
    else:
      out = _tc_final(parts, y, p, dec)
  return out


# speedup vs baseline: 3.9940x; 1.1860x over previous
import functools

import jax
import jax.numpy as jnp
from jax import lax
from jax.experimental import pallas as pl
from jax.experimental.pallas import tpu as pltpu
from jax.experimental.pallas import tpu_sc as plsc

N = 10000
E = 320000
D = 128
DE = 16
NB = 3

NC = 2
NS = 16
L = 16
NW = NC * NS

CH = 128
E_PAD = 327680
EPT = E_PAD // NW
NBLK = EPT // CH
EPS2 = E_PAD // NS
NBLK2 = EPS2 // CH
N_DEN = 10240
NPC = N // NC
SPR = 5128
DUMP = 5120
RPT = 320
_NEG = -1e30

_sc_mesh = plsc.VectorSubcoreMesh(
    core_axis_name="c", subcore_axis_name="s", num_cores=NC, num_subcores=NS
)
_sc_params = pltpu.CompilerParams(needs_layout_passes=False)


def _ln(y, g, b):
  mu = jnp.mean(y, axis=-1, keepdims=True)
  var = jnp.mean((y - mu) ** 2, axis=-1, keepdims=True)
  return (y - mu) / jnp.sqrt(var + 1e-5) * g + b


def _enc_body(x, w1, b1, w2, b2, g, b, wl, bl, wr, br, y_o, xl_o, xr_o):
  h = jnp.maximum(x[...] @ w1[...] + b1[...], 0.0)
  y = h @ w2[...] + b2[...]
  y = _ln(y, g[...], b[...])
  y_o[...] = y
  xl_o[...] = y @ wl[...] + bl[...]
  xr_o[...] = y @ wr[...] + br[...]


def _eproj_body(ea, w0, w1, w2, o0, o1, o2):
  a = ea[...]
  o0[...] = a @ w0[...]
  o1[...] = a @ w1[...]
  o2[...] = a @ w2[...]


def _comb_body(p, y, bias, g, b, wl, bl, wr, br, y_o, xl_o, xr_o):
  o = p[0] + bias[...]
  y2 = y[...] + _ln(o, g[...], b[...])
  y_o[...] = y2
  xl_o[...] = y2 @ wl[...] + bl[...]
  xr_o[...] = y2 @ wr[...] + br[...]


def _final_body(p, y, bias, g, b, w1, b1, w2, b2, out_o):
  o = p[0] + bias[...]
  y2 = y[...] + _ln(o, g[...], b[...])
  h = jnp.maximum(y2 @ w1[...] + b1[...], 0.0)
  out_o[...] = h @ w2[...] + b2[...]


_ROWB = 1000


def _row_spec():
  return pl.BlockSpec((_ROWB, D), lambda i: (i, 0))


def _full_spec(shape):
  n = len(shape)
  return pl.BlockSpec(shape, lambda i: (0,) * n)


def _part_spec():
  return pl.BlockSpec((1, _ROWB, D), lambda i: (i // 5, i % 5, 0))


def _tc_enc(x, enc, gnn0):
  return pl.pallas_call(
      _enc_body,
      grid=(N // _ROWB,),
      in_specs=[
          _row_spec(),
          _full_spec((D, D)), _full_spec((1, D)),
          _full_spec((D, D)), _full_spec((1, D)),
          _full_spec((1, D)), _full_spec((1, D)),
          _full_spec((D, D)), _full_spec((1, D)),
          _full_spec((D, D)), _full_spec((1, D)),
      ],
      out_specs=[_row_spec(), _row_spec(), _row_spec()],
      out_shape=[jax.ShapeDtypeStruct((N, D), jnp.float32)] * 3,
  )(x, enc['W1'], enc['b1'].reshape(1, D), enc['W2'], enc['b2'].reshape(1, D),
    enc['ln_g'].reshape(1, D), enc['ln_b'].reshape(1, D),
    gnn0['Wl'], gnn0['bl'].reshape(1, D), gnn0['Wr'], gnn0['br'].reshape(1, D))


_EB = 4096


def _tc_eproj(ea_pad, we):
  espec = pl.BlockSpec((_EB, DE), lambda i: (i, 0))
  ospec = pl.BlockSpec((_EB, D), lambda i: (i, 0))
  return pl.pallas_call(
      _eproj_body,
      grid=(E_PAD // _EB,),
      in_specs=[espec] + [_full_spec((DE, D))] * 3,
      out_specs=[ospec] * 3,
      out_shape=[jax.ShapeDtypeStruct((E_PAD, D), jnp.float32)] * 3,
  )(ea_pad, we[0], we[1], we[2])


def _tc_comb(parts, y, p_cur, p_nxt):
  return pl.pallas_call(
      _comb_body,
      grid=(N // _ROWB,),
      in_specs=[
          _part_spec(), _row_spec(),
          _full_spec((1, D)), _full_spec((1, D)), _full_spec((1, D)),
          _full_spec((D, D)), _full_spec((1, D)),
          _full_spec((D, D)), _full_spec((1, D)),
      ],
      out_specs=[_row_spec(), _row_spec(), _row_spec()],
      out_shape=[jax.ShapeDtypeStruct((N, D), jnp.float32)] * 3,
  )(parts, y, p_cur['bias'].reshape(1, D), p_cur['ln_g'].reshape(1, D),
    p_cur['ln_b'].reshape(1, D),
    p_nxt['Wl'], p_nxt['bl'].reshape(1, D), p_nxt['Wr'],
    p_nxt['br'].reshape(1, D))


def _tc_final(parts, y, p_cur, dec):
  return pl.pallas_call(
      _final_body,
      grid=(N // _ROWB,),
      in_specs=[
          _part_spec(), _row_spec(),
          _full_spec((1, D)), _full_spec((1, D)), _full_spec((1, D)),
          _full_spec((D, D)), _full_spec((1, D)),
          _full_spec((D, D)), _full_spec((1, D)),
      ],
      out_specs=_row_spec(),
      out_shape=jax.ShapeDtypeStruct((N, D), jnp.float32),
  )(parts, y, p_cur['bias'].reshape(1, D), p_cur['ln_g'].reshape(1, D),
    p_cur['ln_b'].reshape(1, D),
    dec['W1'], dec['b1'].reshape(1, D), dec['W2'], dec['b2'].reshape(1, D))


def _lane_perm(v, idx):
  return jnp.take_along_axis(v, idx, axis=0)


def _bfly_sum(v):
  iota = lax.iota(jnp.int32, L)
  for sh in (8, 4, 2, 1):
    v = v + _lane_perm(v, jnp.bitwise_xor(iota, sh))
  return v


def _bfly_max(v):
  iota = lax.iota(jnp.int32, L)
  for sh in (8, 4, 2, 1):
    v = jnp.maximum(v, _lane_perm(v, jnp.bitwise_xor(iota, sh)))
  return v


def _sc1_body(xl_hbm, xr_hbm, ep_hbm, src_hbm, dst_hbm, att_hbm, logits_hbm,
              src_v, dst_v, att_v,
              sidx0, didx0, xlb0, xrb0, epb0, lb0,
              sidx1, didx1, xlb1, xrb1, epb1, lb1,
              sem0, sem1, lsem0, lsem1):
  cid = lax.axis_index("c")
  sid = lax.axis_index("s")
  tid = sid * NC + cid
  base = tid * EPT
  pltpu.sync_copy(src_hbm.at[pl.ds(base, EPT)], src_v)
  pltpu.sync_copy(dst_hbm.at[pl.ds(base, EPT)], dst_v)
  pltpu.sync_copy(att_hbm, att_v)
  att_c = [att_v[pl.ds(c * L, L)] for c in range(D // L)]
  iota = lax.iota(jnp.int32, L)
  BUFS = ((sidx0, didx0, xlb0, xrb0, epb0, lb0, sem0, lsem0),
          (sidx1, didx1, xlb1, xrb1, epb1, lb1, sem1, lsem1))

  def idx_fire(k, b):
    sx, dx, xlb, xrb, epb, _, sem, _ = BUFS[k]
    off = b * CH
    for g in range(CH // L):
      sx[pl.ds(g * L, L)] = src_v[pl.ds(off + g * L, L)]
      dx[pl.ds(g * L, L)] = dst_v[pl.ds(off + g * L, L)]
    pltpu.async_copy(xl_hbm.at[sx], xlb, sem)
    pltpu.async_copy(xr_hbm.at[dx], xrb, sem)
    pltpu.async_copy(ep_hbm.at[pl.ds(base + off, CH)], epb, sem)

  def drain(k, b):
    sx, dx, xlb, xrb, epb, _, sem, _ = BUFS[k]
    pltpu.make_async_copy(xl_hbm.at[sx], xlb, sem).wait()
    pltpu.make_async_copy(xr_hbm.at[dx], xrb, sem).wait()
    pltpu.make_async_copy(ep_hbm.at[pl.ds(base, CH)], epb, sem).wait()

  def compute(k, b):
    _, _, xlb, xrb, epb, lb, _, lsem = BUFS[k]
    off = b * CH

    @pl.when(b >= 2)
    def _():
      pltpu.make_async_copy(lb, logits_hbm.at[pl.ds(base, CH)], lsem).wait()

    def grp(g, _):
      acc = jnp.zeros((L,), jnp.float32)
      for j in range(L):
        e = g * L + j
        dot = jnp.zeros((L,), jnp.float32)
        for c in range(D // L):
          z = (xlb[e, pl.ds(c * L, L)] + xrb[e, pl.ds(c * L, L)]
               + epb[e, pl.ds(c * L, L)])
          z = jnp.maximum(z, 0.2 * z)
          dot = dot + z * att_c[c]
        svec = _bfly_sum(dot)
        ge = base + off + e
        svec = jnp.where(ge < E, svec, jnp.full((L,), _NEG, jnp.float32))
        acc = jnp.where(iota == j, svec, acc)
      lb[pl.ds(g * L, L)] = acc
      return 0

    lax.fori_loop(0, CH // L, grp, 0)
    pltpu.async_copy(lb, logits_hbm.at[pl.ds(base + off, CH)], lsem)

  idx_fire(0, 0)

  def pair(i, _):
    b0 = i * 2
    idx_fire(1, b0 + 1)
    drain(0, b0)
    compute(0, b0)

    @pl.when(b0 + 2 < NBLK)
    def _():
      idx_fire(0, b0 + 2)

    drain(1, b0 + 1)
    compute(1, b0 + 1)
    return 0

  lax.fori_loop(0, NBLK // 2, pair, 0)
  pltpu.make_async_copy(lb0, logits_hbm.at[pl.ds(base, CH)], lsem0).wait()
  pltpu.make_async_copy(lb1, logits_hbm.at[pl.ds(base, CH)], lsem1).wait()


def _sc1(xl, xr, ep, src_pad, dst_pad, att):
  f = pl.kernel(
      _sc1_body,
      out_type=jax.ShapeDtypeStruct((E_PAD,), jnp.float32),
      mesh=_sc_mesh,
      compiler_params=_sc_params,
      scratch_types=[
          pltpu.VMEM((EPT,), jnp.int32),
          pltpu.VMEM((EPT,), jnp.int32),
          pltpu.VMEM((D,), jnp.float32),
          pltpu.VMEM((CH,), jnp.int32),
          pltpu.VMEM((CH,), jnp.int32),
          pltpu.VMEM((CH, D), jnp.float32),
          pltpu.VMEM((CH, D), jnp.float32),
          pltpu.VMEM((CH, D), jnp.float32),
          pltpu.VMEM((CH,), jnp.float32),
          pltpu.VMEM((CH,), jnp.int32),
          pltpu.VMEM((CH,), jnp.int32),
          pltpu.VMEM((CH, D), jnp.float32),
          pltpu.VMEM((CH, D), jnp.float32),
          pltpu.VMEM((CH, D), jnp.float32),
          pltpu.VMEM((CH,), jnp.float32),
          pltpu.SemaphoreType.DMA,
          pltpu.SemaphoreType.DMA,
          pltpu.SemaphoreType.DMA,
          pltpu.SemaphoreType.DMA,
      ],
  )
  return f(xl, xr, ep, src_pad, dst_pad, att)


CH3 = 64
NB3 = EPS2 // CH3


def _sc2_body(logits_hbm, src_hbm, dst_hbm, xl_hbm, part_hbm,
              den_v, lrange, drange, mxrow, mxbuf,
              exb0, dch0, exb1, dch1,
              sch0, dch20, wb0, xlb0, wrow0,
              sch1, dch21, wb1, xlb1, wrow1,
              dsem0, dsem1, gsem0, gsem1, ssem0, ssem1, isem0, isem1,
              spmem_den, spmem_out, spmem_mx):
  cid = lax.axis_index("c")
  sid = lax.axis_index("s")

  def zrow(i, _):
    for c in range(D // L):
      wrow0[i, pl.ds(c * L, L)] = jnp.zeros((L,), jnp.float32)
    return 0
  lax.fori_loop(0, CH3, zrow, 0)
  for k in range(RPT // CH3):
    pltpu.sync_copy(wrow0, spmem_out.at[pl.ds(sid * RPT + k * CH3, CH3)])

  @pl.when(sid == NS - 1)
  def _():
    pltpu.sync_copy(wrow0.at[pl.ds(0, SPR - NS * RPT)],
                    spmem_out.at[pl.ds(NS * RPT, SPR - NS * RPT)])

  @pl.when(sid == 0)
  def _():
    def zden(i, _):
      den_v[pl.ds(i * L, L)] = jnp.zeros((L,), jnp.float32)
      return 0
    lax.fori_loop(0, N_DEN // L, zden, 0)
    pltpu.sync_copy(den_v, spmem_den)

  plsc.subcore_barrier()

  base2 = sid * EPS2
  pltpu.sync_copy(logits_hbm.at[pl.ds(base2, EPS2)], lrange)
  pltpu.sync_copy(dst_hbm.at[pl.ds(base2, EPS2)], drange)

  def mx_step(g, m):
    return jnp.maximum(m, lrange[pl.ds(g * L, L)])
  m = lax.fori_loop(0, EPS2 // L, mx_step, jnp.full((L,), _NEG, jnp.float32))
  mxrow[pl.ds(0, L)] = m
  pltpu.sync_copy(mxrow, spmem_mx.at[pl.ds(sid * L, L)])
  plsc.subcore_barrier()
  pltpu.sync_copy(spmem_mx, mxbuf)

  def mx_all(i, mm):
    return jnp.maximum(mm, mxbuf[pl.ds(i * L, L)])
  mm = lax.fori_loop(0, NS, mx_all, jnp.full((L,), _NEG, jnp.float32))
  cmax = _bfly_max(mm)

  DBUFS = ((exb0, dch0, dsem0), (exb1, dch1, dsem1))

  def den_one(k, b):
    exb, dch, dsem = DBUFS[k]

    @pl.when(b >= 2)
    def _():
      pltpu.make_async_copy(exb, spmem_den.at[dch], dsem).wait()

    off = b * CH
    for g in range(CH // L):
      lv = lrange[pl.ds(off + g * L, L)]
      exb[pl.ds(g * L, L)] = jnp.exp(lv - cmax)
      dch[pl.ds(g * L, L)] = drange[pl.ds(off + g * L, L)]
    pltpu.async_copy(exb, spmem_den.at[dch], dsem, add=True)

  def den_pair(i, _):
    den_one(0, i * 2)
    den_one(1, i * 2 + 1)
    return 0
  lax.fori_loop(0, NBLK2 // 2, den_pair, 0)
  pltpu.make_async_copy(exb0, spmem_den.at[dch0], dsem0).wait()
  pltpu.make_async_copy(exb1, spmem_den.at[dch1], dsem1).wait()
  plsc.subcore_barrier()

  pltpu.sync_copy(spmem_den, den_v)

  row0 = cid * NPC
  OBUFS = ((sch0, dch20, wb0, xlb0, wrow0, gsem0, ssem0, isem0),
           (sch1, dch21, wb1, xlb1, wrow1, gsem1, ssem1, isem1))

  def idx_fire(k, b):
    sch, isem = OBUFS[k][0], OBUFS[k][7]
    pltpu.async_copy(src_hbm.at[pl.ds(base2 + b * CH3, CH3)], sch, isem)

  def out_one(k, b):
    sch, dch2, wb, xlb, wrow, gsem, ssem, isem = OBUFS[k]
    off = b * CH3

    @pl.when(b >= 2)
    def _():
      pltpu.make_async_copy(wrow, spmem_out.at[dch2], ssem).wait()

    @pl.when(b + 1 < NB3)
    def _():
      osch, oxlb = OBUFS[1 - k][0], OBUFS[1 - k][3]
      pltpu.make_async_copy(src_hbm.at[pl.ds(base2, CH3)], osch,
                            OBUFS[1 - k][7]).wait()
      pltpu.async_copy(xl_hbm.at[osch], oxlb, OBUFS[1 - k][5])

    pltpu.make_async_copy(xl_hbm.at[sch], xlb, gsem).wait()

    @pl.when(b + 2 < NB3)
    def _():
      idx_fire(k, b + 2)

    for g in range(CH3 // L):
      lv = lrange[pl.ds(off + g * L, L)]
      dv = drange[pl.ds(off + g * L, L)]
      denv = plsc.load_gather(den_v, [dv])
      wb[pl.ds(g * L, L)] = jnp.exp(lv - cmax) / (denv + 1e-16)
      loc = dv - row0
      ok = (loc >= 0) & (loc < NPC)
      dch2[pl.ds(g * L, L)] = jnp.where(ok, loc, DUMP)

    def erow_g(g, _):
      wv = wb[pl.ds(g * L, L)]
      for j in range(L):
        e = g * L + j
        w16 = jnp.take_along_axis(wv, jnp.full((L,), j, jnp.int32), axis=0)
        for c in range(D // L):
          wrow[e, pl.ds(c * L, L)] = xlb[e, pl.ds(c * L, L)] * w16
      return 0
    lax.fori_loop(0, CH3 // L, erow_g, 0)
    pltpu.async_copy(wrow, spmem_out.at[dch2], ssem, add=True)

  pltpu.sync_copy(src_hbm.at[pl.ds(base2, CH3)], sch0)
  pltpu.async_copy(xl_hbm.at[sch0], xlb0, gsem0)
  idx_fire(1, 1)

  def out_pair(i, _):
    out_one(0, i * 2)
    out_one(1, i * 2 + 1)
    return 0
  lax.fori_loop(0, NB3 // 2, out_pair, 0)
  pltpu.make_async_copy(wrow0, spmem_out.at[dch20], ssem0).wait()
  pltpu.make_async_copy(wrow1, spmem_out.at[dch21], ssem1).wait()
  plsc.subcore_barrier()

  for k in range(RPT // 64):
    r0 = sid * RPT + k * 64
    pltpu.sync_copy(spmem_out.at[pl.ds(r0, 64)],
                    part_hbm.at[pl.ds(cid * NS * RPT + r0, 64)])


def _sc2(logits, src_pad, dst_pad, xl):
  f = pl.kernel(
      _sc2_body,
      out_type=jax.ShapeDtypeStruct((NC * NS * RPT, D), jnp.float32),
      mesh=_sc_mesh,
      compiler_params=_sc_params,
      scratch_types=[
          pltpu.VMEM((N_DEN,), jnp.float32),
          pltpu.VMEM((EPS2,), jnp.float32),
          pltpu.VMEM((EPS2,), jnp.int32),
          pltpu.VMEM((L,), jnp.float32),
          pltpu.VMEM((NS * L,), jnp.float32),
          pltpu.VMEM((CH,), jnp.float32),
          pltpu.VMEM((CH,), jnp.int32),
          pltpu.VMEM((CH,), jnp.float32),
          pltpu.VMEM((CH,), jnp.int32),
          pltpu.VMEM((CH3,), jnp.int32),
          pltpu.VMEM((CH3,), jnp.int32),
          pltpu.VMEM((CH3,), jnp.float32),
          pltpu.VMEM((CH3, D), jnp.float32),
          pltpu.VMEM((CH3, D), jnp.float32),
          pltpu.VMEM((CH3,), jnp.int32),
          pltpu.VMEM((CH3,), jnp.int32),
          pltpu.VMEM((CH3,), jnp.float32),
          pltpu.VMEM((CH3, D), jnp.float32),
          pltpu.VMEM((CH3, D), jnp.float32),
          pltpu.SemaphoreType.DMA,
          pltpu.SemaphoreType.DMA,
          pltpu.SemaphoreType.DMA,
          pltpu.SemaphoreType.DMA,
          pltpu.SemaphoreType.DMA,
          pltpu.SemaphoreType.DMA,
          pltpu.SemaphoreType.DMA,
          pltpu.SemaphoreType.DMA,
          pltpu.VMEM_SHARED((N_DEN,), jnp.float32),
          pltpu.VMEM_SHARED((SPR, D), jnp.float32),
          pltpu.VMEM_SHARED((NS * L,), jnp.float32),
      ],
  )
  return f(logits, src_pad, dst_pad, xl).reshape(NC, NS * RPT, D)


def kernel(x, edge_index, edge_attr, params):
  src = edge_index[0].astype(jnp.int32)
  dst = edge_index[1].astype(jnp.int32)
  pad = E_PAD - E
  src_pad = jnp.concatenate([src, jnp.zeros((pad,), jnp.int32)])
  dst_pad = jnp.concatenate([dst, jnp.zeros((pad,), jnp.int32)])
  ea_pad = jnp.concatenate(
      [edge_attr, jnp.zeros((pad, DE), jnp.float32)], axis=0)

  enc = params['enc']
  gnn = params['gnn']
  dec = params['dec']

  y, xl, xr = _tc_enc(x, enc, gnn[0])
  eps = _tc_eproj(ea_pad, [gnn[0]['We'], gnn[1]['We'], gnn[2]['We']])

  for i in range(NB):
    p = gnn[i]
    logits = _sc1(xl, xr, eps[i], src_pad, dst_pad, p['att'])
    parts = _sc2(logits, src_pad, dst_pad, xl)
    if i + 1 < NB:
      y, xl, xr = _tc_comb(parts, y, p, gnn[i + 1])
    else:
      out = _tc_final(parts, y, p, dec)
  return out
